# baseline (speedup 1.0000x reference)
import functools

import jax
import jax.numpy as jnp
from jax import lax
from jax.experimental import pallas as pl
from jax.experimental.pallas import tpu as pltpu
from jax.experimental.pallas import tpu_sc as plsc

D = 64
VOCAB = 1000000
B = 16384
NC = 2
NS = 16
LANES = 16
NW = NC * NS
BPW = B // NW
CH = 128
NCH = BPW // CH
GPC = CH // LANES

_mesh = plsc.VectorSubcoreMesh(
    core_axis_name="c", subcore_axis_name="s", num_cores=NC, num_subcores=NS
)


@functools.partial(
    pl.kernel,
    out_type=jax.ShapeDtypeStruct((B * 2,), jnp.float32),
    mesh=_mesh,
    scratch_types=[
        pltpu.VMEM((NCH, CH), jnp.int32),
        pltpu.VMEM((NCH, CH), jnp.int32),
        pltpu.VMEM((CH, 128), jnp.float32),
        pltpu.VMEM((BPW * 2,), jnp.float32),
        pltpu.VMEM((2, D, LANES), jnp.float32),
        pltpu.VMEM((2, LANES), jnp.float32),
        pltpu.SemaphoreType.DMA,
    ],
    compiler_params=pltpu.CompilerParams(needs_layout_passes=False),
)
def _router_sc(tok_hbm, table_hbm, w_hbm, b_hbm, out_hbm,
               raw_v, pidx_v, rows_v, out_v, w_v, b_v, sem):
    wid = lax.axis_index("s") * NC + lax.axis_index("c")

    pltpu.sync_copy(tok_hbm.at[wid], raw_v)
    pltpu.sync_copy(w_hbm, w_v)
    pltpu.sync_copy(b_hbm, b_v)

    iota = lax.iota(jnp.int32, LANES)
    b0 = b_v[0]
    b1 = b_v[1]

    def mk_pidx(i, carry):
        j = i // GPC
        g = i % GPC
        v = raw_v[j, pl.ds(g * LANES, LANES)]
        pidx_v[j, pl.ds(g * LANES, LANES)] = v >> 1
        return carry

    lax.fori_loop(0, NCH * GPC, mk_pidx, 0, unroll=True)

    def chunk(j, carry):
        pltpu.async_copy(table_hbm.at[pidx_v.at[j]], rows_v, sem).wait()

        def group(g, carry2):
            k_idx = g * LANES + iota
            toks = raw_v[j, pl.ds(g * LANES, LANES)]
            c_base = (toks & 1) * D

            def dstep(d, accs):
                a0, a1 = accs
                col = plsc.load_gather(rows_v, [k_idx, c_base + d])
                return (a0 + col * w_v[0, d], a1 + col * w_v[1, d])

            a0, a1 = lax.fori_loop(
                0, D, dstep,
                (jnp.zeros((LANES,), jnp.float32),
                 jnp.zeros((LANES,), jnp.float32)),
                unroll=16,
            )
            out_base = (j * CH + k_idx) * 2
            plsc.store_scatter(out_v, [out_base], a0 + b0)
            plsc.store_scatter(out_v, [out_base + 1], a1 + b1)
            return carry2

        lax.fori_loop(0, GPC, group, 0)
        return carry

    lax.fori_loop(0, NCH, chunk, 0)

    pltpu.sync_copy(out_v, out_hbm.at[pl.ds(wid * BPW * 2, BPW * 2)])


def kernel(ids, token_emb, fc_w, fc_b):
    tok = ids[:, 0].astype(jnp.int32).reshape(NW, NCH, CH)
    table2 = token_emb.reshape(VOCAB // 2, 2 * D)
    w_bcast = jnp.broadcast_to(fc_w[:, :, None], (2, D, LANES))
    b_bcast = jnp.broadcast_to(fc_b[:, None], (2, LANES))
    return _router_sc(tok, table2, w_bcast, b_bcast).reshape(B, 2)


# speedup vs baseline: 1.6439x; 1.6439x over previous
import functools

import jax
import jax.numpy as jnp
from jax import lax
from jax.experimental import pallas as pl
from jax.experimental.pallas import tpu as pltpu
from jax.experimental.pallas import tpu_sc as plsc

D = 64
VOCAB = 1000000
B = 16384
NC = 2
NS = 16
LANES = 16
NW = NC * NS
BPW = B // NW
GROUPS = BPW // LANES

_mesh = plsc.VectorSubcoreMesh(
    core_axis_name="c", subcore_axis_name="s", num_cores=NC, num_subcores=NS
)


@functools.partial(
    pl.kernel,
    out_type=jax.ShapeDtypeStruct((B * 2,), jnp.float32),
    mesh=_mesh,
    scratch_types=[
        pltpu.VMEM((BPW,), jnp.int32),
        pltpu.VMEM((BPW, D), jnp.float32),
        pltpu.VMEM((BPW * 2,), jnp.float32),
        pltpu.VMEM((2, D, LANES), jnp.float32),
        pltpu.VMEM((2, LANES), jnp.float32),
        pltpu.SemaphoreType.DMA,
    ],
    compiler_params=pltpu.CompilerParams(needs_layout_passes=False),
)
def _router_sc(tok_hbm, table_hbm, w_hbm, b_hbm, out_hbm,
               raw_v, rows_v, out_v, w_v, b_v, sem):
    wid = lax.axis_index("s") * NC + lax.axis_index("c")

    pltpu.sync_copy(tok_hbm.at[wid], raw_v)
    pltpu.sync_copy(w_hbm, w_v)
    pltpu.sync_copy(b_hbm, b_v)

    iota = lax.iota(jnp.int32, LANES)
    b0 = b_v[0]
    b1 = b_v[1]

    def fire(g, carry):
        toks = raw_v[pl.ds(g * LANES, LANES)]
        for l in range(LANES):
            t = toks[l]
            pltpu.async_copy(
                table_hbm.at[pl.ds(t, 1)],
                rows_v.at[pl.ds(g * LANES + l, 1)],
                sem,
            )
        return carry

    lax.fori_loop(0, GROUPS, fire, 0)

    def drain(i, carry):
        pltpu.make_async_copy(
            table_hbm.at[pl.ds(0, 1)], rows_v.at[pl.ds(0, 1)], sem
        ).wait()
        return carry

    lax.fori_loop(0, BPW, drain, 0)

    def group(g, carry):
        row_idx = g * LANES + iota

        def dstep(d, accs):
            a0, a1 = accs
            col = plsc.load_gather(rows_v, [row_idx, jnp.full((LANES,), d, jnp.int32)])
            return (a0 + col * w_v[0, d], a1 + col * w_v[1, d])

        a0, a1 = lax.fori_loop(
            0, D, dstep,
            (jnp.zeros((LANES,), jnp.float32),
             jnp.zeros((LANES,), jnp.float32)),
            unroll=16,
        )
        out_base = row_idx * 2
        plsc.store_scatter(out_v, [out_base], a0 + b0)
        plsc.store_scatter(out_v, [out_base + 1], a1 + b1)
        return carry

    lax.fori_loop(0, GROUPS, group, 0)

    pltpu.sync_copy(out_v, out_hbm.at[pl.ds(wid * BPW * 2, BPW * 2)])


def kernel(ids, token_emb, fc_w, fc_b):
    tok = ids[:, 0].astype(jnp.int32).reshape(NW, BPW)
    w_bcast = jnp.broadcast_to(fc_w[:, :, None], (2, D, LANES))
    b_bcast = jnp.broadcast_to(fc_b[:, None], (2, LANES))
    return _router_sc(tok, token_emb, w_bcast, b_bcast).reshape(B, 2)


# speedup vs baseline: 1.6883x; 1.0270x over previous
import functools

import jax
import jax.numpy as jnp
from jax import lax
from jax.experimental import pallas as pl
from jax.experimental.pallas import tpu as pltpu
from jax.experimental.pallas import tpu_sc as plsc

D = 64
VOCAB = 1000000
B = 16384
P = 8
NC = 2
NS = 16
LANES = 16
NW = NC * NS
BPW = B // NW
GROUPS = BPW // LANES

TC_BLK = 4096
TC_GRID = (VOCAB + TC_BLK - 1) // TC_BLK


def _project_body(t_ref, w_ref, b_ref, o_ref):
    o_ref[...] = (
        lax.dot_general(
            t_ref[...], w_ref[...],
            dimension_numbers=(((0,), (0,)), ((), ())),
            preferred_element_type=jnp.float32,
        )
        + b_ref[...]
    )


_project_tc = pl.pallas_call(
    _project_body,
    grid=(TC_GRID,),
    in_specs=[
        pl.BlockSpec((D, TC_BLK), lambda i: (0, i)),
        pl.BlockSpec((D, P), lambda i: (0, 0)),
        pl.BlockSpec((1, P), lambda i: (0, 0)),
    ],
    out_specs=pl.BlockSpec((TC_BLK, P), lambda i: (i, 0)),
    out_shape=jax.ShapeDtypeStruct((VOCAB, P), jnp.float32),
)


_mesh = plsc.VectorSubcoreMesh(
    core_axis_name="c", subcore_axis_name="s", num_cores=NC, num_subcores=NS
)


@functools.partial(
    pl.kernel,
    out_type=jax.ShapeDtypeStruct((B * 2,), jnp.float32),
    mesh=_mesh,
    scratch_types=[
        pltpu.VMEM((BPW,), jnp.int32),
        pltpu.VMEM((BPW, P), jnp.float32),
        pltpu.VMEM((BPW * 2,), jnp.float32),
        pltpu.SemaphoreType.DMA,
    ],
    compiler_params=pltpu.CompilerParams(needs_layout_passes=False),
)
def _gather_sc(tok_hbm, p8_hbm, out_hbm, raw_v, rows_v, out_v, sem):
    wid = lax.axis_index("s") * NC + lax.axis_index("c")

    pltpu.sync_copy(tok_hbm.at[wid], raw_v)

    iota = lax.iota(jnp.int32, LANES)
    zeros_i = jnp.zeros((LANES,), jnp.int32)
    ones_i = jnp.full((LANES,), 1, jnp.int32)

    def fire(g, carry):
        toks = raw_v[pl.ds(g * LANES, LANES)]
        for l in range(LANES):
            t = toks[l]
            pltpu.async_copy(
                p8_hbm.at[pl.ds(t, 1)],
                rows_v.at[pl.ds(g * LANES + l, 1)],
                sem,
            )
        return carry

    lax.fori_loop(0, GROUPS, fire, 0)

    def drain(i, carry):
        pltpu.make_async_copy(
            p8_hbm.at[pl.ds(0, 1)], rows_v.at[pl.ds(0, 1)], sem
        ).wait()
        return carry

    lax.fori_loop(0, BPW, drain, 0)

    def group(g, carry):
        row_idx = g * LANES + iota
        a0 = plsc.load_gather(rows_v, [row_idx, zeros_i])
        a1 = plsc.load_gather(rows_v, [row_idx, ones_i])
        out_base = row_idx * 2
        plsc.store_scatter(out_v, [out_base], a0)
        plsc.store_scatter(out_v, [out_base + 1], a1)
        return carry

    lax.fori_loop(0, GROUPS, group, 0)

    pltpu.sync_copy(out_v, out_hbm.at[pl.ds(wid * BPW * 2, BPW * 2)])


def kernel(ids, token_emb, fc_w, fc_b):
    tok = ids[:, 0].astype(jnp.int32).reshape(NW, BPW)
    table_t = token_emb.T
    w8 = jnp.zeros((D, P), jnp.float32).at[:, :2].set(fc_w.T)
    b8 = jnp.zeros((1, P), jnp.float32).at[0, :2].set(fc_b)
    p8 = _project_tc(table_t, w8, b8)
    return _gather_sc(tok, p8).reshape(B, 2)


# speedup vs baseline: 2.2040x; 1.3055x over previous
import functools

import jax
import jax.numpy as jnp
from jax import lax
from jax.experimental import pallas as pl
from jax.experimental.pallas import tpu as pltpu
from jax.experimental.pallas import tpu_sc as plsc

D = 64
VOCAB = 1000000
B = 16384
P = 8
NC = 2
NS = 16
LANES = 16
NW = NC * NS
BPW = B // NW
GROUPS = BPW // LANES

CHW = 512
TLIM = (VOCAB // 128) * 128
NTAIL = VOCAB - TLIM
NCHUNK_ALL = TLIM // CHW
CPW = -(-NCHUNK_ALL // NW)
BLKS = CHW // (8 * LANES)

_mesh = plsc.VectorSubcoreMesh(
    core_axis_name="c", subcore_axis_name="s", num_cores=NC, num_subcores=NS
)


@functools.partial(
    pl.kernel,
    out_type=jax.ShapeDtypeStruct((VOCAB * P,), jnp.float32),
    mesh=_mesh,
    scratch_types=[
        pltpu.VMEM((D, CHW), jnp.float32),
        pltpu.VMEM((D, CHW), jnp.float32),
        pltpu.VMEM((CHW * P,), jnp.float32),
        pltpu.VMEM((D, NTAIL), jnp.float32),
        pltpu.VMEM((2, D, LANES), jnp.float32),
        pltpu.VMEM((2, LANES), jnp.float32),
        pltpu.SemaphoreType.DMA,
        pltpu.SemaphoreType.DMA,
    ],
    compiler_params=pltpu.CompilerParams(needs_layout_passes=False),
)
def _project_sc(table_hbm, tail_hbm, w_hbm, b_hbm, p8_hbm,
                buf_a, buf_b, stage_v, tail_v, w_v, b_v, sem_a, sem_b):
    wid = lax.axis_index("s") * NC + lax.axis_index("c")

    pltpu.sync_copy(w_hbm, w_v)
    pltpu.sync_copy(b_hbm, b_v)

    iota = lax.iota(jnp.int32, LANES)
    zeros16 = jnp.zeros((LANES,), jnp.float32)
    b0 = b_v[0]
    b1 = b_v[1]

    def chunk_off(i):
        g = jnp.minimum(wid * CPW + i, NCHUNK_ALL - 1)
        return pl.multiple_of(g * CHW, 128)

    def fire(i, buf, sem):
        pltpu.async_copy(table_hbm.at[:, pl.ds(chunk_off(i), CHW)], buf, sem)

    def wait_chunk(buf, sem):
        pltpu.make_async_copy(table_hbm.at[:, pl.ds(0, CHW)], buf, sem).wait()

    def project_chunk(i, buf):
        off = chunk_off(i)

        def block(blk, carry):
            base = blk * 8 * LANES

            def dstep(d, accs):
                w0 = w_v[0, d]
                w1 = w_v[1, d]
                new = []
                for g8 in range(8):
                    col = buf[d, pl.ds(base + g8 * LANES, LANES)]
                    new.append(accs[2 * g8] + col * w0)
                    new.append(accs[2 * g8 + 1] + col * w1)
                return tuple(new)

            accs = lax.fori_loop(0, D, dstep, (zeros16,) * 16, unroll=8)
            for g8 in range(8):
                tok = base + g8 * LANES + iota
                plsc.store_scatter(stage_v, [tok * P], accs[2 * g8] + b0)
                plsc.store_scatter(stage_v, [tok * P + 1], accs[2 * g8 + 1] + b1)
            return carry

        lax.fori_loop(0, BLKS, block, 0)
        pltpu.sync_copy(stage_v, p8_hbm.at[pl.ds(off * P, CHW * P)])

    fire(0, buf_a, sem_a)
    fire(1, buf_b, sem_b)

    def pairbody(k, carry):
        i = 2 * k
        wait_chunk(buf_a, sem_a)
        project_chunk(i, buf_a)
        fire(i + 2, buf_a, sem_a)
        wait_chunk(buf_b, sem_b)
        project_chunk(i + 1, buf_b)
        fire(i + 3, buf_b, sem_b)
        return carry

    lax.fori_loop(0, CPW // 2, pairbody, 0)
    wait_chunk(buf_a, sem_a)
    wait_chunk(buf_b, sem_b)

    @pl.when(wid == 0)
    def _():
        pltpu.sync_copy(tail_hbm, tail_v)

        def dstep_t(d, accs):
            w0 = w_v[0, d]
            w1 = w_v[1, d]
            new = []
            for g8 in range(4):
                col = tail_v[d, pl.ds(g8 * LANES, LANES)]
                new.append(accs[2 * g8] + col * w0)
                new.append(accs[2 * g8 + 1] + col * w1)
            return tuple(new)

        accs = lax.fori_loop(0, D, dstep_t, (zeros16,) * 8, unroll=8)
        for g8 in range(4):
            tok = g8 * LANES + iota
            plsc.store_scatter(stage_v, [tok * P], accs[2 * g8] + b0)
            plsc.store_scatter(stage_v, [tok * P + 1], accs[2 * g8 + 1] + b1)
        pltpu.sync_copy(stage_v.at[pl.ds(0, NTAIL * P)],
                        p8_hbm.at[pl.ds(TLIM * P, NTAIL * P)])


@functools.partial(
    pl.kernel,
    out_type=jax.ShapeDtypeStruct((B * 2,), jnp.float32),
    mesh=_mesh,
    scratch_types=[
        pltpu.VMEM((BPW,), jnp.int32),
        pltpu.VMEM((BPW * P,), jnp.float32),
        pltpu.VMEM((BPW * 2,), jnp.float32),
        pltpu.SemaphoreType.DMA,
    ],
    compiler_params=pltpu.CompilerParams(needs_layout_passes=False),
)
def _gather_sc(tok_hbm, p8_hbm, out_hbm, raw_v, rows_v, out_v, sem):
    wid = lax.axis_index("s") * NC + lax.axis_index("c")

    pltpu.sync_copy(tok_hbm.at[wid], raw_v)

    iota = lax.iota(jnp.int32, LANES)
    zeros_i = jnp.zeros((LANES,), jnp.int32)
    ones_i = jnp.full((LANES,), 1, jnp.int32)

    def fire(g, carry):
        toks = raw_v[pl.ds(g * LANES, LANES)]
        for l in range(LANES):
            t = toks[l]
            pltpu.async_copy(
                p8_hbm.at[pl.ds(t * P, P)],
                rows_v.at[pl.ds((g * LANES + l) * P, P)],
                sem,
            )
        return carry

    lax.fori_loop(0, GROUPS, fire, 0)

    def drain(i, carry):
        pltpu.make_async_copy(
            p8_hbm.at[pl.ds(0, P)], rows_v.at[pl.ds(0, P)], sem
        ).wait()
        return carry

    lax.fori_loop(0, BPW, drain, 0)

    def group(g, carry):
        row_idx = g * LANES + iota
        a0 = plsc.load_gather(rows_v, [row_idx * P])
        a1 = plsc.load_gather(rows_v, [row_idx * P + 1])
        out_base = row_idx * 2
        plsc.store_scatter(out_v, [out_base], a0)
        plsc.store_scatter(out_v, [out_base + 1], a1)
        return carry

    lax.fori_loop(0, GROUPS, group, 0)

    pltpu.sync_copy(out_v, out_hbm.at[pl.ds(wid * BPW * 2, BPW * 2)])


def kernel(ids, token_emb, fc_w, fc_b):
    tok = ids[:, 0].astype(jnp.int32).reshape(NW, BPW)
    table_t = token_emb.T
    tail_t = table_t[:, TLIM:]
    w_bcast = jnp.broadcast_to(fc_w[:, :, None], (2, D, LANES))
    b_bcast = jnp.broadcast_to(fc_b[:, None], (2, LANES))
    p8 = _project_sc(table_t, tail_t, w_bcast, b_bcast)
    return _gather_sc(tok, p8).reshape(B, 2)
